# Initial kernel scaffold; baseline (speedup 1.0000x reference)
import jax
import jax.numpy as jnp
from jax.experimental import pallas as pl


def kernel(inputs, w0, w, v):
    raise NotImplementedError("write your pallas kernel here")


# speedup vs baseline: 8.1182x; 8.1182x over previous
import functools

import jax
import jax.numpy as jnp
from jax import lax
from jax.experimental import pallas as pl
from jax.experimental.pallas import tpu as pltpu
from jax.experimental.pallas import tpu_sc as plsc

FIELDS = 26
FEAT_PER_FIELD = 10000
TOTAL_FEAT = FIELDS * FEAT_PER_FIELD
KDIM = 8
ROWW = FIELDS * KDIM
BATCH = 4096
NWORKERS = 32
ROWS_PER_W = BATCH // NWORKERS
ROWS_PER_CHUNK = 4
CHUNKS = ROWS_PER_W // ROWS_PER_CHUNK
LOOK = ROWS_PER_CHUNK * FIELDS
NV = ROWW // 16


def _ffm_body(inp_hbm, v_hbm, w_hbm, out_hbm,
              idx0, idx1, rows0, rows1, wr0, wr1, outv,
              sv0, sv1, sw0, sw1):
    cid = lax.axis_index("c")
    sid = lax.axis_index("s")
    wid = sid * 2 + cid
    wflat = wid * (ROWS_PER_W * FIELDS)
    io = lax.iota(jnp.int32, 16)
    permi = lax.rem(io + 8, 16)
    lane0 = io == 0
    idxb = (idx0, idx1)
    rowsb = (rows0, rows1)
    wrb = (wr0, wr1)
    semv = (sv0, sv1)
    semw = (sw0, sw1)

    def prep(cc, p):
        start = pl.multiple_of(wflat + cc * LOOK, 8)
        pltpu.sync_copy(inp_hbm.at[pl.ds(start, LOOK)], idxb[p])
        for m in range(6):
            sl = pl.ds(m * 16, 16)
            offs = lax.rem(io + (m * 16) % FIELDS, FIELDS) * FEAT_PER_FIELD
            idxb[p][sl] = idxb[p][sl] + offs
        sl = pl.ds(88, 16)
        offs = jnp.where(io >= 8,
                         lax.rem(io + 10, FIELDS) * FEAT_PER_FIELD, 0)
        idxb[p][sl] = idxb[p][sl] + offs
        pltpu.make_async_copy(v_hbm.at[idxb[p]], rowsb[p], semv[p]).start()
        pltpu.make_async_copy(w_hbm.at[idxb[p]], wrb[p], semw[p]).start()

    def compute(cc, p):
        pltpu.make_async_copy(v_hbm.at[idxb[p]], rowsb[p], semv[p]).wait()
        pltpu.make_async_copy(w_hbm.at[idxb[p]], wrb[p], semw[p]).wait()
        zero = jnp.zeros((16,), jnp.float32)
        for r in range(ROWS_PER_CHUNK):

            def one_field(row, S16, Q16):
                acc = rowsb[p][row, pl.ds(0, 16)]
                for m in range(1, NV):
                    acc = acc + rowsb[p][row, pl.ds(m * 16, 16)]
                u16 = acc + acc.at[permi].get(mode="promise_in_bounds")
                return S16 + u16, Q16 + u16 * u16

            def fbody(h, carry):
                S16, Q16 = carry
                base = r * FIELDS + h * 2
                S16, Q16 = one_field(base, S16, Q16)
                S16, Q16 = one_field(base + 1, S16, Q16)
                return S16, Q16

            S16, Q16 = lax.fori_loop(0, FIELDS // 2, fbody, (zero, zero))
            s2 = jnp.sum(S16 * S16)
            qs = jnp.sum(Q16)
            w1i = r * FIELDS + io
            w2i = jnp.where(io < 10, r * FIELDS + 16 + io, 0)
            g1 = plsc.load_gather(wrb[p], [w1i])
            g2 = plsc.load_gather(wrb[p], [w2i], mask=io < 10)
            wsum = jnp.sum(g1 + jnp.where(io < 10, g2, 0.0))
            val = wsum + 0.25 * (s2 - qs)
            vid = jnp.full((16,), cc * ROWS_PER_CHUNK + r, jnp.int32)
            plsc.store_scatter(outv, [vid],
                               jnp.full((16,), val, jnp.float32), mask=lane0)

    prep(0, 0)

    def loop_body(i, carry):
        for p in range(2):
            cc = i * 2 + p

            @pl.when(cc + 1 < CHUNKS)
            def _():
                prep(cc + 1, (p + 1) % 2)

            compute(cc, p)
        return carry

    lax.fori_loop(0, CHUNKS // 2, loop_body, 0)
    obase = pl.multiple_of(wid * ROWS_PER_W, 8)
    pltpu.sync_copy(outv, out_hbm.at[pl.ds(obase, ROWS_PER_W)])


@jax.jit
def _ffm(inp_flat, v2, w):
    mesh = plsc.VectorSubcoreMesh(core_axis_name="c", subcore_axis_name="s")
    fn = pl.kernel(
        _ffm_body,
        mesh=mesh,
        compiler_params=pltpu.CompilerParams(
            needs_layout_passes=False, use_tc_tiling_on_sc=False),
        out_type=jax.ShapeDtypeStruct((BATCH,), jnp.float32),
        scratch_types=[
            pltpu.VMEM((LOOK,), jnp.int32),
            pltpu.VMEM((LOOK,), jnp.int32),
            pltpu.VMEM((LOOK, ROWW), jnp.float32),
            pltpu.VMEM((LOOK, ROWW), jnp.float32),
            pltpu.VMEM((LOOK,), jnp.float32),
            pltpu.VMEM((LOOK,), jnp.float32),
            pltpu.VMEM((ROWS_PER_W,), jnp.float32),
            pltpu.SemaphoreType.DMA,
            pltpu.SemaphoreType.DMA,
            pltpu.SemaphoreType.DMA,
            pltpu.SemaphoreType.DMA,
        ],
    )
    return fn(inp_flat, v2, w)


def kernel(inputs, w0, w, v):
    inp_flat = inputs.reshape(-1)
    v2 = v.reshape(TOTAL_FEAT, ROWW)
    out = _ffm(inp_flat, v2, w.reshape(-1))
    return out.reshape(BATCH, 1) + w0
